# baseline (speedup 1.0000x reference)
import functools

import jax
import jax.numpy as jnp
from jax import lax
from jax.experimental import pallas as pl
from jax.experimental.pallas import tpu as pltpu
from jax.experimental.pallas import tpu_sc as plsc

S, B, D = 250, 128, 1024
NC, NS = 2, 16
NW = NC * NS
UMAX = (S + NW - 1) // NW
CROWS = 64
NCHUNK = B // CROWS
NSLICE = D // 16


def _sc_body(x_hbm, pe_hbm, out_hbm, pe_v, buf_v, sem):
    wid = lax.axis_index("s") * NC + lax.axis_index("c")

    def unit(u, _):
        s = wid + u * NW

        @pl.when(s < S)
        def _():
            pltpu.sync_copy(pe_hbm.at[pl.ds(s, 1), :], pe_v)
            for c in range(NCHUNK):
                b0 = c * CROWS
                pltpu.sync_copy(x_hbm.at[s, pl.ds(b0, CROWS), :], buf_v)
                for j in range(NSLICE):
                    pe_vec = pe_v[0, pl.ds(j * 16, 16)]

                    def row(i, _):
                        cur = buf_v[i, pl.ds(j * 16, 16)]
                        buf_v[i, pl.ds(j * 16, 16)] = cur + pe_vec
                        return 0

                    lax.fori_loop(0, CROWS, row, 0)
                pltpu.sync_copy(buf_v, out_hbm.at[s, pl.ds(b0, CROWS), :])

        return 0

    lax.fori_loop(0, UMAX, unit, 0)


def kernel(x, pos_embed):
    mesh = plsc.VectorSubcoreMesh(core_axis_name="c", subcore_axis_name="s")
    k = functools.partial(
        pl.kernel,
        mesh=mesh,
        out_type=jax.ShapeDtypeStruct((S, B, D), jnp.float32),
        scratch_types=[
            pltpu.VMEM((1, D), jnp.float32),
            pltpu.VMEM((CROWS, D), jnp.float32),
            pltpu.SemaphoreType.DMA,
        ],
    )(_sc_body)
    return k(x, pos_embed)


# speedup vs baseline: 3.5093x; 3.5093x over previous
import functools

import jax
import jax.numpy as jnp
from jax import lax
from jax.experimental import pallas as pl
from jax.experimental.pallas import tpu as pltpu
from jax.experimental.pallas import tpu_sc as plsc

S, B, D = 250, 128, 1024
NC, NS = 2, 16
NW = NC * NS
UMAX = (S + NW - 1) // NW
CROWS = 32
CPU_ = B // CROWS
NCK_MAX = UMAX * CPU_
NSLICE = D // 16
RUNROLL = 8
NBUF = 3


def _chunk_src(x_hbm, wid, kk):
    s = wid + (kk >> 2) * NW
    b0 = (kk & 3) * CROWS
    return x_hbm.at[s, pl.ds(b0, CROWS), :]


def _chunk_dst(out_hbm, wid, kk):
    s = wid + (kk >> 2) * NW
    b0 = (kk & 3) * CROWS
    return out_hbm.at[s, pl.ds(b0, CROWS), :]


def _sc_body(x_hbm, pe_hbm, out_hbm, pe_v, b0_v, b1_v, b2_v,
             si0, si1, si2, so0, so1, so2):
    wid = lax.axis_index("s") * NC + lax.axis_index("c")
    bufs = (b0_v, b1_v, b2_v)
    sin = (si0, si1, si2)
    sout = (so0, so1, so2)
    nck = jnp.where(wid < S - (UMAX - 1) * NW, NCK_MAX, NCK_MAX - CPU_)

    pltpu.async_copy(_chunk_src(x_hbm, wid, 0), b0_v, si0)
    pltpu.async_copy(_chunk_src(x_hbm, wid, 1), b1_v, si1)

    def slot(g, _):
        for b in range(NBUF):
            kk = g * NBUF + b
            buf = bufs[b]
            b2i = (b + 2) % NBUF

            @pl.when(kk < nck)
            def _():
                @pl.when((kk & 3) == 0)
                def _():
                    pltpu.sync_copy(
                        pe_hbm.at[pl.ds(wid + (kk >> 2) * NW, 1), :], pe_v)

                pltpu.make_async_copy(
                    _chunk_src(x_hbm, wid, kk), buf, sin[b]).wait()

                @pl.when(kk + 2 < nck)
                def _():
                    @pl.when(kk >= 1)
                    def _():
                        pltpu.make_async_copy(
                            bufs[b2i], _chunk_dst(out_hbm, wid, kk - 1),
                            sout[b2i]).wait()

                    pltpu.async_copy(
                        _chunk_src(x_hbm, wid, kk + 2), bufs[b2i], sin[b2i])

                def grp(g2, _):
                    i0 = g2 * RUNROLL

                    def jstep(j, _):
                        pe_vec = pe_v[0, pl.ds(j * 16, 16)]
                        for r in range(RUNROLL):
                            i = i0 + r
                            buf[i, pl.ds(j * 16, 16)] = (
                                buf[i, pl.ds(j * 16, 16)] + pe_vec)
                        return 0

                    lax.fori_loop(0, NSLICE, jstep, 0)
                    return 0

                lax.fori_loop(0, CROWS // RUNROLL, grp, 0)

                pltpu.async_copy(buf, _chunk_dst(out_hbm, wid, kk), sout[b])

        return 0

    lax.fori_loop(0, NCK_MAX // NBUF + 1, slot, 0)

    for b in range(NBUF):
        kk_b = nck - 3 + ((b - nck) % NBUF + NBUF) % NBUF
        pltpu.make_async_copy(
            bufs[b], _chunk_dst(out_hbm, wid, kk_b), sout[b]).wait()


def kernel(x, pos_embed):
    mesh = plsc.VectorSubcoreMesh(core_axis_name="c", subcore_axis_name="s")
    k = functools.partial(
        pl.kernel,
        mesh=mesh,
        out_type=jax.ShapeDtypeStruct((S, B, D), jnp.float32),
        scratch_types=[
            pltpu.VMEM((1, D), jnp.float32),
            pltpu.VMEM((CROWS, D), jnp.float32),
            pltpu.VMEM((CROWS, D), jnp.float32),
            pltpu.VMEM((CROWS, D), jnp.float32),
            pltpu.SemaphoreType.DMA,
            pltpu.SemaphoreType.DMA,
            pltpu.SemaphoreType.DMA,
            pltpu.SemaphoreType.DMA,
            pltpu.SemaphoreType.DMA,
            pltpu.SemaphoreType.DMA,
        ],
    )(_sc_body)
    return k(x, pos_embed)


# speedup vs baseline: 3.7670x; 1.0734x over previous
import functools

import jax
import jax.numpy as jnp
from jax import lax
from jax.experimental import pallas as pl
from jax.experimental.pallas import tpu as pltpu
from jax.experimental.pallas import tpu_sc as plsc

S, B, D = 250, 128, 1024
NC, NS = 2, 16
NW = NC * NS
UMAX = (S + NW - 1) // NW
CROWS = 32
CPU_ = B // CROWS
NCK_MAX = UMAX * CPU_
NSLICE = D // 16
RUNROLL = 8
NBUF = 3


def _chunk_src(x_hbm, wid, kk):
    s = wid + (kk >> 2) * NW
    b0 = (kk & 3) * CROWS
    return x_hbm.at[s, pl.ds(b0, CROWS), :]


def _chunk_dst(out_hbm, wid, kk):
    s = wid + (kk >> 2) * NW
    b0 = (kk & 3) * CROWS
    return out_hbm.at[s, pl.ds(b0, CROWS), :]


def _sc_body(x_hbm, pe_hbm, out_hbm, pe_v, b0_v, b1_v, b2_v,
             si0, si1, si2, so0, so1, so2):
    wid = lax.axis_index("s") * NC + lax.axis_index("c")
    bufs = (b0_v, b1_v, b2_v)
    sin = (si0, si1, si2)
    sout = (so0, so1, so2)
    nck = jnp.where(wid < S - (UMAX - 1) * NW, NCK_MAX, NCK_MAX - CPU_)

    pltpu.async_copy(_chunk_src(x_hbm, wid, 0), b0_v, si0)
    pltpu.async_copy(_chunk_src(x_hbm, wid, 1), b1_v, si1)

    def slot(g, _):
        for b in range(NBUF):
            kk = g * NBUF + b
            buf = bufs[b]
            b2i = (b + 2) % NBUF

            @pl.when(kk < nck)
            def _():
                @pl.when((kk & 3) == 0)
                def _():
                    pltpu.sync_copy(
                        pe_hbm.at[pl.ds(wid + (kk >> 2) * NW, 1), :], pe_v)

                pltpu.make_async_copy(
                    _chunk_src(x_hbm, wid, kk), buf, sin[b]).wait()

                @pl.when(kk + 2 < nck)
                def _():
                    @pl.when(kk >= 1)
                    def _():
                        pltpu.make_async_copy(
                            bufs[b2i], _chunk_dst(out_hbm, wid, kk - 1),
                            sout[b2i]).wait()

                    pltpu.async_copy(
                        _chunk_src(x_hbm, wid, kk + 2), bufs[b2i], sin[b2i])

                def grp(g2, _):
                    i0 = g2 * RUNROLL

                    def jstep(j, _):
                        pe_vec = pe_v[0, pl.ds(j * 16, 16)]
                        for r in range(RUNROLL):
                            plsc.addupdate(
                                buf.at[i0 + r, pl.ds(j * 16, 16)], pe_vec)
                        return 0

                    lax.fori_loop(0, NSLICE, jstep, 0)
                    return 0

                lax.fori_loop(0, CROWS // RUNROLL, grp, 0)

                pltpu.async_copy(buf, _chunk_dst(out_hbm, wid, kk), sout[b])

        return 0

    lax.fori_loop(0, NCK_MAX // NBUF + 1, slot, 0)

    for b in range(NBUF):
        kk_b = nck - 3 + ((b - nck) % NBUF + NBUF) % NBUF
        pltpu.make_async_copy(
            bufs[b], _chunk_dst(out_hbm, wid, kk_b), sout[b]).wait()


def kernel(x, pos_embed):
    mesh = plsc.VectorSubcoreMesh(core_axis_name="c", subcore_axis_name="s")
    k = functools.partial(
        pl.kernel,
        mesh=mesh,
        out_type=jax.ShapeDtypeStruct((S, B, D), jnp.float32),
        scratch_types=[
            pltpu.VMEM((1, D), jnp.float32),
            pltpu.VMEM((CROWS, D), jnp.float32),
            pltpu.VMEM((CROWS, D), jnp.float32),
            pltpu.VMEM((CROWS, D), jnp.float32),
            pltpu.SemaphoreType.DMA,
            pltpu.SemaphoreType.DMA,
            pltpu.SemaphoreType.DMA,
            pltpu.SemaphoreType.DMA,
            pltpu.SemaphoreType.DMA,
            pltpu.SemaphoreType.DMA,
        ],
    )(_sc_body)
    return k(x, pos_embed)


# speedup vs baseline: 4.1450x; 1.1004x over previous
import functools

import jax
import jax.numpy as jnp
from jax import lax
from jax.experimental import pallas as pl
from jax.experimental.pallas import tpu as pltpu
from jax.experimental.pallas import tpu_sc as plsc

S, B, D = 250, 128, 1024
NC, NS = 2, 16
NW = NC * NS
UMAX = (S + NW - 1) // NW
CROWS = 32
CPU_ = B // CROWS
NCK_MAX = UMAX * CPU_
NSLICE = D // 16
RUNROLL = 8
NBUF = 3


def _chunk_src(x_hbm, wid, kk):
    s = wid + (kk >> 2) * NW
    b0 = (kk & 3) * CROWS
    return x_hbm.at[s, pl.ds(b0, CROWS), :]


def _chunk_dst(out_hbm, wid, kk):
    s = wid + (kk >> 2) * NW
    b0 = (kk & 3) * CROWS
    return out_hbm.at[s, pl.ds(b0, CROWS), :]


def _sc_body(x_hbm, pe_hbm, out_hbm, pe_v, b0_v, b1_v, b2_v,
             si0, si1, si2, so0, so1, so2):
    wid = lax.axis_index("s") * NC + lax.axis_index("c")
    bufs = (b0_v, b1_v, b2_v)
    sin = (si0, si1, si2)
    sout = (so0, so1, so2)
    nck = jnp.where(wid < S - (UMAX - 1) * NW, NCK_MAX, NCK_MAX - CPU_)

    pltpu.async_copy(_chunk_src(x_hbm, wid, 0), b0_v, si0)
    pltpu.async_copy(_chunk_src(x_hbm, wid, 1), b1_v, si1)

    def slot(g, _):
        for b in range(NBUF):
            kk = g * NBUF + b
            buf = bufs[b]
            b2i = (b + 2) % NBUF

            @pl.when(kk < nck)
            def _():
                @pl.when((kk & 3) == 0)
                def _():
                    pltpu.sync_copy(
                        pe_hbm.at[pl.ds(wid + (kk >> 2) * NW, 1), :], pe_v)

                pltpu.make_async_copy(
                    _chunk_src(x_hbm, wid, kk), buf, sin[b]).wait()

                @pl.when(kk + 2 < nck)
                def _():
                    @pl.when(kk >= 1)
                    def _():
                        pltpu.make_async_copy(
                            bufs[b2i], _chunk_dst(out_hbm, wid, kk - 1),
                            sout[b2i]).wait()

                    pltpu.async_copy(
                        _chunk_src(x_hbm, wid, kk + 2), bufs[b2i], sin[b2i])

                def jstep(j, _):
                    sl = pl.ds(j * 16, 16)
                    pe_vec = pe_v[0, sl]
                    for r in range(CROWS):
                        plsc.addupdate(buf.at[r, sl], pe_vec)
                    return 0

                lax.fori_loop(0, NSLICE, jstep, 0)

                pltpu.async_copy(buf, _chunk_dst(out_hbm, wid, kk), sout[b])

        return 0

    lax.fori_loop(0, NCK_MAX // NBUF + 1, slot, 0)

    for b in range(NBUF):
        kk_b = nck - 3 + ((b - nck) % NBUF + NBUF) % NBUF
        pltpu.make_async_copy(
            bufs[b], _chunk_dst(out_hbm, wid, kk_b), sout[b]).wait()


def kernel(x, pos_embed):
    mesh = plsc.VectorSubcoreMesh(core_axis_name="c", subcore_axis_name="s")
    k = functools.partial(
        pl.kernel,
        mesh=mesh,
        out_type=jax.ShapeDtypeStruct((S, B, D), jnp.float32),
        scratch_types=[
            pltpu.VMEM((1, D), jnp.float32),
            pltpu.VMEM((CROWS, D), jnp.float32),
            pltpu.VMEM((CROWS, D), jnp.float32),
            pltpu.VMEM((CROWS, D), jnp.float32),
            pltpu.SemaphoreType.DMA,
            pltpu.SemaphoreType.DMA,
            pltpu.SemaphoreType.DMA,
            pltpu.SemaphoreType.DMA,
            pltpu.SemaphoreType.DMA,
            pltpu.SemaphoreType.DMA,
        ],
    )(_sc_body)
    return k(x, pos_embed)
